# initial kernel scaffold (unmeasured)
import jax
import jax.numpy as jnp
from jax import lax
from jax.experimental import pallas as pl
from jax.experimental.pallas import tpu as pltpu

N_GLOBAL = 2048
EPS = 1e-5


def kernel(x, gamma):
    m, n = x.shape
    rows_s, rows_l = m // 128, 128

    def body(x_ref, g_ref, out_ref, comm_ref, send_sem, recv_sem):
        my_x = lax.axis_index("x")
        my_y = lax.axis_index("y")
        partner = (my_x, 1 - my_y)

        xf = x_ref[:, :].astype(jnp.float32)
        partial = jnp.sum(xf * xf, axis=1)
        comm_ref[0, :, :] = partial.reshape(rows_s, rows_l)

        barrier = pltpu.get_barrier_semaphore()
        pl.semaphore_signal(
            barrier, inc=1, device_id=partner,
            device_id_type=pl.DeviceIdType.MESH,
        )
        pl.semaphore_wait(barrier, 1)

        rdma = pltpu.make_async_remote_copy(
            src_ref=comm_ref.at[0],
            dst_ref=comm_ref.at[1],
            send_sem=send_sem,
            recv_sem=recv_sem,
            device_id=partner,
            device_id_type=pl.DeviceIdType.MESH,
        )
        rdma.start()
        rdma.wait()

        total = comm_ref[0, :, :] + comm_ref[1, :, :]
        inv = lax.rsqrt(total / N_GLOBAL + EPS)
        inv_col = inv.reshape(m, 1)
        g = g_ref[:, :].astype(jnp.float32)
        out_ref[:, :] = (xf * inv_col * g).astype(jnp.bfloat16)

    return pl.pallas_call(
        body,
        out_shape=jax.ShapeDtypeStruct((m, n), jnp.bfloat16),
        in_specs=[
            pl.BlockSpec(memory_space=pltpu.VMEM),
            pl.BlockSpec(memory_space=pltpu.VMEM),
        ],
        out_specs=pl.BlockSpec(memory_space=pltpu.VMEM),
        scratch_shapes=[
            pltpu.VMEM((2, rows_s, rows_l), jnp.float32),
            pltpu.SemaphoreType.DMA,
            pltpu.SemaphoreType.DMA,
        ],
        compiler_params=pltpu.CompilerParams(collective_id=0),
    )(x, gamma.reshape(1, n))


# baseline (device time: 45436 ns/iter reference)
import jax
import jax.numpy as jnp
from jax import lax
from jax.experimental import pallas as pl
from jax.experimental.pallas import tpu as pltpu

N_GLOBAL = 2048
EPS = 1e-5


def kernel(x, gamma):
    m, n = x.shape

    def body(x_ref, g_ref, out_ref, comm_ref, send_sem, recv_sem):
        my_x = lax.axis_index("x")
        my_y = lax.axis_index("y")
        partner = (my_x, 1 - my_y)

        xf = x_ref[:, :].astype(jnp.float32)
        partial = jnp.sum(xf * xf, axis=1, keepdims=True)
        comm_ref[0, :, :] = partial

        barrier = pltpu.get_barrier_semaphore()
        pl.semaphore_signal(
            barrier, inc=1, device_id=partner,
            device_id_type=pl.DeviceIdType.MESH,
        )
        pl.semaphore_wait(barrier, 1)

        rdma = pltpu.make_async_remote_copy(
            src_ref=comm_ref.at[0],
            dst_ref=comm_ref.at[1],
            send_sem=send_sem,
            recv_sem=recv_sem,
            device_id=partner,
            device_id_type=pl.DeviceIdType.MESH,
        )
        rdma.start()
        rdma.wait()

        total = comm_ref[0, :, :] + comm_ref[1, :, :]
        inv = lax.rsqrt(total / N_GLOBAL + EPS)
        g = g_ref[:, :].astype(jnp.float32)
        out_ref[:, :] = (xf * inv * g).astype(jnp.bfloat16)

    return pl.pallas_call(
        body,
        out_shape=jax.ShapeDtypeStruct((m, n), jnp.bfloat16),
        in_specs=[
            pl.BlockSpec(memory_space=pltpu.VMEM),
            pl.BlockSpec(memory_space=pltpu.VMEM),
        ],
        out_specs=pl.BlockSpec(memory_space=pltpu.VMEM),
        scratch_shapes=[
            pltpu.VMEM((2, m, 1), jnp.float32),
            pltpu.SemaphoreType.DMA,
            pltpu.SemaphoreType.DMA,
        ],
        compiler_params=pltpu.CompilerParams(
            collective_id=0,
            vmem_limit_bytes=100 * 1024 * 1024,
        ),
    )(x, gamma.reshape(1, n))


# device time: 23782 ns/iter; 1.9105x vs baseline; 1.9105x over previous
import jax
import jax.numpy as jnp
from jax import lax
from jax.experimental import pallas as pl
from jax.experimental.pallas import tpu as pltpu

N_GLOBAL = 2048
EPS = 1e-5


def kernel(x, gamma):
    m, n = x.shape
    mb = m // 128

    def body(x_ref, g_ref, out_ref, comm_ref, send_sem, recv_sem):
        my_x = lax.axis_index("x")
        my_y = lax.axis_index("y")
        partner = (my_x, 1 - my_y)

        row_q = lax.broadcasted_iota(jnp.int32, (m, mb), 0)
        col_i = lax.broadcasted_iota(jnp.int32, (m, mb), 1)
        A = (row_q // 128 == col_i).astype(jnp.float32)
        row_q2 = lax.broadcasted_iota(jnp.int32, (m, 128), 0)
        col_j = lax.broadcasted_iota(jnp.int32, (m, 128), 1)
        E = (row_q2 % 128 == col_j).astype(jnp.float32)

        xf = x_ref[:, :].astype(jnp.float32)
        partial = jnp.sum(xf * xf, axis=1, keepdims=True)
        packed = jax.lax.dot_general(
            A, partial * E,
            (((0,), (0,)), ((), ())),
            preferred_element_type=jnp.float32,
        )
        comm_ref[0, :, :] = packed

        barrier = pltpu.get_barrier_semaphore()
        pl.semaphore_signal(
            barrier, inc=1, device_id=partner,
            device_id_type=pl.DeviceIdType.MESH,
        )
        pl.semaphore_wait(barrier, 1)

        rdma = pltpu.make_async_remote_copy(
            src_ref=comm_ref.at[0],
            dst_ref=comm_ref.at[1],
            send_sem=send_sem,
            recv_sem=recv_sem,
            device_id=partner,
            device_id_type=pl.DeviceIdType.MESH,
        )
        rdma.start()
        rdma.wait()

        total = comm_ref[0, :, :] + comm_ref[1, :, :]
        inv = lax.rsqrt(total / N_GLOBAL + EPS)
        spread = jax.lax.dot_general(
            A, inv,
            (((1,), (0,)), ((), ())),
            preferred_element_type=jnp.float32,
        )
        inv_col = jnp.sum(spread * E, axis=1, keepdims=True)
        g = g_ref[:, :].astype(jnp.float32)
        out_ref[:, :] = (xf * inv_col * g).astype(jnp.bfloat16)

    return pl.pallas_call(
        body,
        out_shape=jax.ShapeDtypeStruct((m, n), jnp.bfloat16),
        in_specs=[
            pl.BlockSpec(memory_space=pltpu.VMEM),
            pl.BlockSpec(memory_space=pltpu.VMEM),
        ],
        out_specs=pl.BlockSpec(memory_space=pltpu.VMEM),
        scratch_shapes=[
            pltpu.VMEM((2, mb, 128), jnp.float32),
            pltpu.SemaphoreType.DMA,
            pltpu.SemaphoreType.DMA,
        ],
        compiler_params=pltpu.CompilerParams(
            collective_id=0,
            vmem_limit_bytes=100 * 1024 * 1024,
        ),
    )(x, gamma.reshape(1, n))
